# Initial kernel scaffold; baseline (speedup 1.0000x reference)
import jax
import jax.numpy as jnp
from jax.experimental import pallas as pl


def kernel(seeds, es, neighbors, w_ih, w_hh, b_ih, b_hh):
    raise NotImplementedError("write your pallas kernel here")


# speedup vs baseline: 3.8517x; 3.8517x over previous
import jax
import jax.numpy as jnp
from jax import lax
from jax.experimental import pallas as pl
from jax.experimental.pallas import tpu as pltpu

_N = 8192
_D = 256
_NCLS = 8
_K = 16
_STEPS = 3
_NEG_INF = float("-inf")


def _gru(x, h, w_ih, w_hh, b_ih, b_hh):
    gi = x @ w_ih.T + b_ih
    gh = h @ w_hh.T + b_hh
    i_r, i_z, i_n = jnp.split(gi, 3, axis=1)
    h_r, h_z, h_n = jnp.split(gh, 3, axis=1)
    r = jax.nn.sigmoid(i_r + h_r)
    z = jax.nn.sigmoid(i_z + h_z)
    n = jnp.tanh(i_n + r * h_n)
    return (1.0 - z) * n + z * h


def _cos(a, b):
    a = a / (jnp.linalg.norm(a, axis=-1, keepdims=True) + 1e-8)
    b = b / (jnp.linalg.norm(b, axis=-1, keepdims=True) + 1e-8)
    return a @ b.T


def _cnt_body(nb_ref, cm_ref, out_ref):
    @pl.when(pl.program_id(1) == 0)
    def _zero():
        out_ref[...] = jnp.zeros_like(out_ref)

    out_ref[...] += lax.dot_general(
        nb_ref[...], cm_ref[...].astype(jnp.float32),
        (((1,), (0,)), ((), ())), preferred_element_type=jnp.float32)


def _count_matvec(neighbors, cmask):
    br, bk = 512, 2048
    return pl.pallas_call(
        _cnt_body,
        grid=(_N // br, _N // bk),
        in_specs=[
            pl.BlockSpec((br, bk), lambda r, k: (r, k)),
            pl.BlockSpec((bk, _NCLS), lambda r, k: (k, 0)),
        ],
        out_specs=pl.BlockSpec((br, _NCLS), lambda r, k: (r, 0)),
        out_shape=jax.ShapeDtypeStruct((_N, _NCLS), jnp.float32),
    )(neighbors, cmask)


def _select_body(sims_ref, cnt_ref, em_ref, cm_ref,
                 selidx_ref, out2_ref, em_new_ref, cm_new_ref,
                 scores_ref, probs_ref, avail_ref):
    sims = sims_ref[...]
    valid = (cnt_ref[...] >= 2.0) & (em_ref[...] == 0)
    scores_ref[...] = jnp.where(valid, sims, _NEG_INF)
    pools = jnp.any(valid, axis=1, keepdims=True)
    probs_ref[...] = jnp.where(pools, sims, 0.0)
    avail_ref[...] = jnp.ones_like(avail_ref)
    iota = lax.broadcasted_iota(jnp.int32, (_N, _NCLS), 0)
    for k in range(_K):
        av = avail_ref[...] != 0
        sa = jnp.where(av, scores_ref[...], _NEG_INF)
        m = jnp.max(sa, axis=0, keepdims=True)
        elig = (sa == m) & av
        idxr = jnp.min(jnp.where(elig, iota, _N), axis=0, keepdims=True)
        oh = iota == idxr
        avail_ref[...] = jnp.where(oh, 0, avail_ref[...])
        selidx_ref[pl.ds(k, 1), :] = idxr
        out2_ref[pl.ds(k * _NCLS, _NCLS), :] = lax.dot_general(
            oh.astype(jnp.float32), probs_ref[...],
            (((0,), (0,)), ((), ())), preferred_element_type=jnp.float32)
    picked = 1 - avail_ref[...]
    cm_new_ref[...] = cm_ref[...] | picked
    em_new_ref[...] = em_ref[...] | jnp.max(picked, axis=1, keepdims=True)


def _select_call(sims, cnt, em, cm):
    f32 = jnp.float32
    i32 = jnp.int32
    return pl.pallas_call(
        _select_body,
        grid=(1,),
        in_specs=[
            pl.BlockSpec((_N, _NCLS), lambda i: (0, 0)),
            pl.BlockSpec((_N, _NCLS), lambda i: (0, 0)),
            pl.BlockSpec((_N, 1), lambda i: (0, 0)),
            pl.BlockSpec((_N, _NCLS), lambda i: (0, 0)),
        ],
        out_specs=[
            pl.BlockSpec((_K, _NCLS), lambda i: (0, 0)),
            pl.BlockSpec((_K * _NCLS, _NCLS), lambda i: (0, 0)),
            pl.BlockSpec((_N, 1), lambda i: (0, 0)),
            pl.BlockSpec((_N, _NCLS), lambda i: (0, 0)),
        ],
        out_shape=[
            jax.ShapeDtypeStruct((_K, _NCLS), i32),
            jax.ShapeDtypeStruct((_K * _NCLS, _NCLS), f32),
            jax.ShapeDtypeStruct((_N, 1), i32),
            jax.ShapeDtypeStruct((_N, _NCLS), i32),
        ],
        scratch_shapes=[
            pltpu.VMEM((_N, _NCLS), f32),
            pltpu.VMEM((_N, _NCLS), f32),
            pltpu.VMEM((_N, _NCLS), i32),
        ],
    )(sims, cnt, em, cm)


def kernel(seeds, es, neighbors, w_ih, w_hh, b_ih, b_hh):
    seeds = seeds.astype(jnp.int32)
    i32 = jnp.int32
    kcls = jnp.arange(_NCLS * _K, dtype=i32) // _K
    em = jnp.zeros((_N,), i32).at[seeds].set(1).reshape(_N, 1)
    cm = jnp.zeros((_N, _NCLS), i32).at[seeds, kcls].set(1)
    hx = jnp.zeros((_NCLS, _D), jnp.float32)
    last = seeds
    outs, sels, hxs = [], [], []
    for _ in range(_STEPS):
        inp = es[last].reshape(_NCLS, -1, _D).mean(axis=1)
        hx = _gru(inp, hx, w_ih, w_hh, b_ih, b_hh)
        hxs.append(hx)
        sims = _cos(es, hx) * 0.5 + 0.5
        cnt = _count_matvec(neighbors, cm)
        selidx, out2, em, cm = _select_call(sims, cnt, em, cm)
        sel = selidx.T.reshape(_NCLS * _K)
        outs.append(out2.reshape(_K, _NCLS, _NCLS).transpose(1, 0, 2)
                    .reshape(_NCLS * _K, _NCLS))
        sels.append(sel)
        last = sel
    return jnp.stack(outs), jnp.stack(sels), jnp.stack(hxs)
